# baseline (speedup 1.0000x reference)
import functools

import jax
import jax.numpy as jnp
from jax import lax
from jax.experimental import pallas as pl
from jax.experimental.pallas import tpu as pltpu
from jax.experimental.pallas import tpu_sc as plsc


def _pack_body(w_ref, x_ref, b_ref, pk_ref):
    r = lax.dot_general(
        w_ref[...], x_ref[...],
        dimension_numbers=(((1,), (1,)), ((), ())),
        preferred_element_type=jnp.float32,
    )
    u0 = lax.bitcast_convert_type(r[0] + b_ref[0, 0], jnp.uint32)
    u1 = lax.bitcast_convert_type(r[1], jnp.uint32)
    half = jnp.uint32(0x8000)
    pk = ((u0 + half) & jnp.uint32(0xFFFF0000)) | ((u1 + half) >> 16)
    pk_ref[...] = lax.bitcast_convert_type(pk, jnp.int32)


def kernel(x, edge_index, edge_weight, W, b):
    n, d = x.shape
    e = edge_index.shape[1]

    wt = W.reshape(2, d)
    bias = b.astype(jnp.float32).reshape(1, 1)
    ei = edge_index.astype(jnp.int32)
    ew = edge_weight.astype(jnp.float32)

    nblk_tc = 2048
    n_pad = ((n + nblk_tc - 1) // nblk_tc) * nblk_tc
    steps = n_pad // nblk_tc

    pk = pl.pallas_call(
        _pack_body,
        grid=(steps,),
        in_specs=[
            pl.BlockSpec((2, d), lambda i: (0, 0)),
            pl.BlockSpec((nblk_tc, d), lambda i: (i, 0)),
            pl.BlockSpec((1, 1), lambda i: (0, 0)),
        ],
        out_specs=pl.BlockSpec((nblk_tc,), lambda i: (i,)),
        out_shape=jax.ShapeDtypeStruct((n_pad,), jnp.int32),
    )(wt, x, bias)

    info = plsc.get_sparse_core_info()
    nc, ns, lanes = info.num_cores, info.num_subcores, info.num_lanes
    nw = nc * ns
    blk = 128
    nblk = e // blk
    nb_lo = nblk // nw
    n_hi = nblk - nb_lo * nw
    l_hi = (nb_lo + 1) * blk

    @functools.partial(
        pl.kernel,
        mesh=plsc.VectorSubcoreMesh(core_axis_name="c", subcore_axis_name="s"),
        out_type=jax.ShapeDtypeStruct((e,), jnp.float32),
        compiler_params=pltpu.CompilerParams(needs_layout_passes=False,
                                             use_tc_tiling_on_sc=True),
        scratch_types=[
            pltpu.VMEM((n_pad,), jnp.int32),
            pltpu.VMEM((2, l_hi), jnp.int32),
            pltpu.VMEM((l_hi,), jnp.float32),
            pltpu.VMEM((l_hi,), jnp.float32),
            pltpu.SemaphoreType.DMA,
            pltpu.SemaphoreType.DMA,
        ],
    )
    def edge_scores(pk_hbm, ei_hbm, ew_hbm, out_hbm,
                    pk_v, ei_v, ew_v, out_v, sem, sem_out):
        cid = lax.axis_index("c")
        sid = lax.axis_index("s")
        wid = sid * nc + cid

        hi_mask = jnp.int32(-65536)

        def compute(lo, hi):
            @plsc.parallel_loop(lo, hi, step=lanes, unroll=8)
            def body(off):
                s = ei_v[0, pl.ds(off, lanes)]
                t = ei_v[1, pl.ds(off, lanes)]
                g1 = plsc.load_gather(pk_v, [s])
                g2 = plsc.load_gather(pk_v, [t])
                p1 = plsc.bitcast(g1 & hi_mask, jnp.float32)
                p2 = plsc.bitcast(g2 << 16, jnp.float32)
                z = p1 + p2
                sig = 1.0 / (1.0 + jnp.exp(-z))
                out_v[pl.ds(off, lanes)] = ew_v[pl.ds(off, lanes)] * sig

        def run(nb):
            nba = (nb + 1) // 2
            la = nba * blk
            lb = (nb - nba) * blk
            c0 = pl.multiple_of(
                (nb_lo * wid + jnp.minimum(wid, n_hi)) * blk, blk)
            copies_a = [
                pltpu.async_copy(pk_hbm, pk_v, sem),
                pltpu.async_copy(ei_hbm.at[:, pl.ds(c0, la)],
                                 ei_v.at[:, pl.ds(0, la)], sem),
                pltpu.async_copy(ew_hbm.at[pl.ds(c0, la)],
                                 ew_v.at[pl.ds(0, la)], sem),
            ]
            copies_b = [
                pltpu.async_copy(ei_hbm.at[:, pl.ds(c0 + la, lb)],
                                 ei_v.at[:, pl.ds(la, lb)], sem),
                pltpu.async_copy(ew_hbm.at[pl.ds(c0 + la, lb)],
                                 ew_v.at[pl.ds(la, lb)], sem),
            ]
            for c in copies_a:
                c.wait()
            compute(0, la)
            out_a = pltpu.async_copy(out_v.at[pl.ds(0, la)],
                                     out_hbm.at[pl.ds(c0, la)], sem_out)
            for c in copies_b:
                c.wait()
            compute(la, la + lb)
            out_b = pltpu.async_copy(out_v.at[pl.ds(la, lb)],
                                     out_hbm.at[pl.ds(c0 + la, lb)], sem_out)
            out_a.wait()
            out_b.wait()

        @pl.when(wid < n_hi)
        def _():
            run(nb_lo + 1)

        @pl.when(wid >= n_hi)
        def _():
            run(nb_lo)

    return edge_scores(pk, ei, ew)


# speedup vs baseline: 1.0162x; 1.0162x over previous
import functools

import jax
import jax.numpy as jnp
from jax import lax
from jax.experimental import pallas as pl
from jax.experimental.pallas import tpu as pltpu
from jax.experimental.pallas import tpu_sc as plsc


def _pack_body(w_ref, x_ref, b_ref, pk_ref):
    r = lax.dot_general(
        w_ref[...], x_ref[...],
        dimension_numbers=(((1,), (1,)), ((), ())),
        preferred_element_type=jnp.float32,
    )
    u0 = lax.bitcast_convert_type(r[0] + b_ref[0, 0], jnp.uint32)
    u1 = lax.bitcast_convert_type(r[1], jnp.uint32)
    half = jnp.uint32(0x8000)
    pk = ((u0 + half) & jnp.uint32(0xFFFF0000)) | ((u1 + half) >> 16)
    pk_ref[...] = lax.bitcast_convert_type(pk, jnp.int32)


def kernel(x, edge_index, edge_weight, W, b):
    n, d = x.shape
    e = edge_index.shape[1]

    wt = W.reshape(2, d)
    bias = b.astype(jnp.float32).reshape(1, 1)
    ei = edge_index.astype(jnp.int32)
    ew = edge_weight.astype(jnp.float32)

    n_pad = n

    pk = pl.pallas_call(
        _pack_body,
        out_shape=jax.ShapeDtypeStruct((n_pad,), jnp.int32),
    )(wt, x, bias)

    info = plsc.get_sparse_core_info()
    nc, ns, lanes = info.num_cores, info.num_subcores, info.num_lanes
    nw = nc * ns
    blk = 128
    nblk = e // blk
    nb_lo = nblk // nw
    n_hi = nblk - nb_lo * nw
    l_hi = (nb_lo + 1) * blk

    @functools.partial(
        pl.kernel,
        mesh=plsc.VectorSubcoreMesh(core_axis_name="c", subcore_axis_name="s"),
        out_type=jax.ShapeDtypeStruct((e,), jnp.float32),
        compiler_params=pltpu.CompilerParams(needs_layout_passes=False,
                                             use_tc_tiling_on_sc=True),
        scratch_types=[
            pltpu.VMEM((n_pad,), jnp.int32),
            pltpu.VMEM((2, l_hi), jnp.int32),
            pltpu.VMEM((l_hi,), jnp.float32),
            pltpu.VMEM((l_hi,), jnp.float32),
            pltpu.SemaphoreType.DMA,
            pltpu.SemaphoreType.DMA,
        ],
    )
    def edge_scores(pk_hbm, ei_hbm, ew_hbm, out_hbm,
                    pk_v, ei_v, ew_v, out_v, sem, sem_out):
        cid = lax.axis_index("c")
        sid = lax.axis_index("s")
        wid = sid * nc + cid

        hi_mask = jnp.int32(-65536)

        def compute(lo, hi):
            @plsc.parallel_loop(lo, hi, step=lanes, unroll=8)
            def body(off):
                s = ei_v[0, pl.ds(off, lanes)]
                t = ei_v[1, pl.ds(off, lanes)]
                g1 = plsc.load_gather(pk_v, [s])
                g2 = plsc.load_gather(pk_v, [t])
                p1 = plsc.bitcast(g1 & hi_mask, jnp.float32)
                p2 = plsc.bitcast(g2 << 16, jnp.float32)
                z = p1 + p2
                sig = 1.0 / (1.0 + jnp.exp(-z))
                out_v[pl.ds(off, lanes)] = ew_v[pl.ds(off, lanes)] * sig

        pk_copy = pltpu.async_copy(pk_hbm, pk_v, sem)

        def run(nb):
            qs = [nb // 4 + (1 if q < nb % 4 else 0) for q in range(4)]
            c0 = pl.multiple_of(
                (nb_lo * wid + jnp.minimum(wid, n_hi)) * blk, blk)
            copies = []
            off = 0
            for q in qs:
                lq = q * blk
                copies.append((off, lq, [
                    pltpu.async_copy(ei_hbm.at[:, pl.ds(c0 + off, lq)],
                                     ei_v.at[:, pl.ds(off, lq)], sem),
                    pltpu.async_copy(ew_hbm.at[pl.ds(c0 + off, lq)],
                                     ew_v.at[pl.ds(off, lq)], sem),
                ]))
                off += lq
            pk_copy.wait()
            outs = []
            for off, lq, cs in copies:
                for c in cs:
                    c.wait()
                compute(off, off + lq)
                outs.append(pltpu.async_copy(
                    out_v.at[pl.ds(off, lq)],
                    out_hbm.at[pl.ds(c0 + off, lq)], sem_out))
            for o in outs:
                o.wait()

        @pl.when(wid < n_hi)
        def _():
            run(nb_lo + 1)

        @pl.when(wid >= n_hi)
        def _():
            run(nb_lo)

    return edge_scores(pk, ei, ew)


# speedup vs baseline: 1.0466x; 1.0299x over previous
import functools

import jax
import jax.numpy as jnp
from jax import lax
from jax.experimental import pallas as pl
from jax.experimental.pallas import tpu as pltpu
from jax.experimental.pallas import tpu_sc as plsc


def _pack_body(w_ref, x_ref, b_ref, pk_ref):
    r = lax.dot_general(
        w_ref[...], x_ref[...],
        dimension_numbers=(((1,), (1,)), ((), ())),
        preferred_element_type=jnp.float32,
    )
    u0 = lax.bitcast_convert_type(r[0] + b_ref[0, 0], jnp.uint32)
    u1 = lax.bitcast_convert_type(r[1], jnp.uint32)
    half = jnp.uint32(0x8000)
    pk = ((u0 + half) & jnp.uint32(0xFFFF0000)) | ((u1 + half) >> 16)
    pk_ref[...] = lax.bitcast_convert_type(pk, jnp.int32)


def kernel(x, edge_index, edge_weight, W, b):
    n, d = x.shape
    e = edge_index.shape[1]

    wt = W.reshape(2, d)
    bias = b.astype(jnp.float32).reshape(1, 1)
    ei = edge_index.astype(jnp.int32)
    ew = edge_weight.astype(jnp.float32)

    n_pad = n

    pk = pl.pallas_call(
        _pack_body,
        out_shape=jax.ShapeDtypeStruct((n_pad,), jnp.int32),
    )(wt, x, bias)

    info = plsc.get_sparse_core_info()
    nc, ns, lanes = info.num_cores, info.num_subcores, info.num_lanes
    nw = nc * ns
    blk = 128
    nblk = e // blk
    nb_lo = nblk // nw
    n_hi = nblk - nb_lo * nw
    l_hi = (nb_lo + 1) * blk

    @functools.partial(
        pl.kernel,
        mesh=plsc.VectorSubcoreMesh(core_axis_name="c", subcore_axis_name="s"),
        out_type=jax.ShapeDtypeStruct((e,), jnp.float32),
        compiler_params=pltpu.CompilerParams(needs_layout_passes=False,
                                             use_tc_tiling_on_sc=True),
        scratch_types=[
            pltpu.VMEM((n_pad,), jnp.int32),
            pltpu.VMEM((2, l_hi), jnp.int32),
            pltpu.VMEM((l_hi,), jnp.float32),
            pltpu.VMEM((l_hi,), jnp.float32),
            pltpu.SemaphoreType.DMA,
            pltpu.SemaphoreType.DMA,
        ],
    )
    def edge_scores(pk_hbm, ei_hbm, ew_hbm, out_hbm,
                    pk_v, ei_v, ew_v, out_v, sem, sem_out):
        cid = lax.axis_index("c")
        sid = lax.axis_index("s")
        wid = sid * nc + cid

        hi_mask = jnp.int32(-65536)

        def compute(lo, hi):
            @plsc.parallel_loop(lo, hi, step=lanes, unroll=8)
            def body(off):
                s = ei_v[0, pl.ds(off, lanes)]
                t = ei_v[1, pl.ds(off, lanes)]
                g1 = plsc.load_gather(pk_v, [s])
                g2 = plsc.load_gather(pk_v, [t])
                p1 = plsc.bitcast(g1 & hi_mask, jnp.float32)
                p2 = plsc.bitcast(g2 << 16, jnp.float32)
                z = p1 + p2
                sig = 1.0 / (1.0 + jnp.exp(-z))
                out_v[pl.ds(off, lanes)] = ew_v[pl.ds(off, lanes)] * sig

        l_lo = nb_lo * blk
        c0 = pl.multiple_of(wid * l_lo, blk)
        copies = [
            pltpu.async_copy(pk_hbm, pk_v, sem),
            pltpu.async_copy(ei_hbm.at[:, pl.ds(c0, l_lo)],
                             ei_v.at[:, pl.ds(0, l_lo)], sem),
            pltpu.async_copy(ew_hbm.at[pl.ds(c0, l_lo)],
                             ew_v.at[pl.ds(0, l_lo)], sem),
        ]

        tail0 = nw * l_lo

        ct = pl.multiple_of(tail0 + wid * blk, blk)

        @pl.when(wid < n_hi)
        def _():
            pltpu.async_copy(ei_hbm.at[:, pl.ds(ct, blk)],
                             ei_v.at[:, pl.ds(l_lo, blk)], sem)
            pltpu.async_copy(ew_hbm.at[pl.ds(ct, blk)],
                             ew_v.at[pl.ds(l_lo, blk)], sem)

        for c in copies:
            c.wait()
        compute(0, l_lo)
        out_main = pltpu.async_copy(out_v.at[pl.ds(0, l_lo)],
                                    out_hbm.at[pl.ds(c0, l_lo)], sem_out)

        @pl.when(wid < n_hi)
        def _():
            pltpu.make_async_copy(ei_hbm.at[:, pl.ds(ct, blk)],
                                  ei_v.at[:, pl.ds(l_lo, blk)], sem).wait()
            pltpu.make_async_copy(ew_hbm.at[pl.ds(ct, blk)],
                                  ew_v.at[pl.ds(l_lo, blk)], sem).wait()
            compute(l_lo, l_lo + blk)
            pltpu.async_copy(out_v.at[pl.ds(l_lo, blk)],
                             out_hbm.at[pl.ds(ct, blk)], sem_out).wait()

        out_main.wait()

    return edge_scores(pk, ei, ew)


# speedup vs baseline: 1.0583x; 1.0112x over previous
import functools

import jax
import jax.numpy as jnp
from jax import lax
from jax.experimental import pallas as pl
from jax.experimental.pallas import tpu as pltpu
from jax.experimental.pallas import tpu_sc as plsc


def _pack_body(w_ref, x_ref, b_ref, pk_ref):
    r = lax.dot_general(
        w_ref[...], x_ref[...],
        dimension_numbers=(((1,), (1,)), ((), ())),
        preferred_element_type=jnp.float32,
    )
    u0 = lax.bitcast_convert_type(r[0] + b_ref[0, 0], jnp.uint32)
    u1 = lax.bitcast_convert_type(r[1], jnp.uint32)
    half = jnp.uint32(0x8000)
    pk = ((u0 + half) & jnp.uint32(0xFFFF0000)) | ((u1 + half) >> 16)
    pk_ref[...] = lax.bitcast_convert_type(pk, jnp.int32)


def kernel(x, edge_index, edge_weight, W, b):
    n, d = x.shape
    e = edge_index.shape[1]

    wt = W.reshape(2, d)
    bias = b.astype(jnp.float32).reshape(1, 1)
    ei = edge_index.astype(jnp.int32)
    ew = edge_weight.astype(jnp.float32)

    n_pad = n

    pk = pl.pallas_call(
        _pack_body,
        out_shape=jax.ShapeDtypeStruct((n_pad,), jnp.int32),
    )(wt, x, bias)

    info = plsc.get_sparse_core_info()
    nc, ns, lanes = info.num_cores, info.num_subcores, info.num_lanes
    nw = nc * ns
    blk = 128
    nblk = e // blk
    nb_lo = nblk // nw
    n_hi = nblk - nb_lo * nw
    l_hi = (nb_lo + 1) * blk

    @functools.partial(
        pl.kernel,
        mesh=plsc.VectorSubcoreMesh(core_axis_name="c", subcore_axis_name="s"),
        out_type=jax.ShapeDtypeStruct((e,), jnp.float32),
        compiler_params=pltpu.CompilerParams(needs_layout_passes=False,
                                             use_tc_tiling_on_sc=True),
        scratch_types=[
            pltpu.VMEM((n_pad,), jnp.int32),
            pltpu.VMEM((2, l_hi), jnp.int32),
            pltpu.VMEM((l_hi,), jnp.float32),
            pltpu.VMEM((l_hi,), jnp.float32),
            pltpu.SemaphoreType.DMA,
            pltpu.SemaphoreType.DMA,
        ],
    )
    def edge_scores(pk_hbm, ei_hbm, ew_hbm, out_hbm,
                    pk_v, ei_v, ew_v, out_v, sem, sem_out):
        cid = lax.axis_index("c")
        sid = lax.axis_index("s")
        wid = sid * nc + cid

        hi_mask = jnp.int32(-65536)

        def compute(lo, hi):
            @plsc.parallel_loop(lo, hi, step=lanes, unroll=8)
            def body(off):
                s = ei_v[0, pl.ds(off, lanes)]
                t = ei_v[1, pl.ds(off, lanes)]
                g1 = plsc.load_gather(pk_v, [s])
                g2 = plsc.load_gather(pk_v, [t])
                p1 = plsc.bitcast(g1 & hi_mask, jnp.float32)
                p2 = plsc.bitcast(g2 << 16, jnp.float32)
                z = p1 + p2
                sig = 1.0 / (1.0 + jnp.exp(-z))
                out_v[pl.ds(off, lanes)] = ew_v[pl.ds(off, lanes)] * sig

        l_lo = nb_lo * blk
        la = (nb_lo // 2) * blk
        lb = l_lo - la
        c0 = pl.multiple_of(wid * l_lo, blk)
        copies_a = [
            pltpu.async_copy(pk_hbm, pk_v, sem),
            pltpu.async_copy(ei_hbm.at[:, pl.ds(c0, la)],
                             ei_v.at[:, pl.ds(0, la)], sem),
            pltpu.async_copy(ew_hbm.at[pl.ds(c0, la)],
                             ew_v.at[pl.ds(0, la)], sem),
        ]
        copies_b = [
            pltpu.async_copy(ei_hbm.at[:, pl.ds(c0 + la, lb)],
                             ei_v.at[:, pl.ds(la, lb)], sem),
            pltpu.async_copy(ew_hbm.at[pl.ds(c0 + la, lb)],
                             ew_v.at[pl.ds(la, lb)], sem),
        ]

        tail0 = nw * l_lo

        ct = pl.multiple_of(tail0 + wid * blk, blk)

        @pl.when(wid < n_hi)
        def _():
            pltpu.async_copy(ei_hbm.at[:, pl.ds(ct, blk)],
                             ei_v.at[:, pl.ds(l_lo, blk)], sem)
            pltpu.async_copy(ew_hbm.at[pl.ds(ct, blk)],
                             ew_v.at[pl.ds(l_lo, blk)], sem)

        for c in copies_a:
            c.wait()
        compute(0, la)
        out_a = pltpu.async_copy(out_v.at[pl.ds(0, la)],
                                 out_hbm.at[pl.ds(c0, la)], sem_out)
        for c in copies_b:
            c.wait()
        compute(la, l_lo)
        out_main = pltpu.async_copy(out_v.at[pl.ds(la, lb)],
                                    out_hbm.at[pl.ds(c0 + la, lb)], sem_out)

        @pl.when(wid < n_hi)
        def _():
            pltpu.make_async_copy(ei_hbm.at[:, pl.ds(ct, blk)],
                                  ei_v.at[:, pl.ds(l_lo, blk)], sem).wait()
            pltpu.make_async_copy(ew_hbm.at[pl.ds(ct, blk)],
                                  ew_v.at[pl.ds(l_lo, blk)], sem).wait()
            compute(l_lo, l_lo + blk)
            pltpu.async_copy(out_v.at[pl.ds(l_lo, blk)],
                             out_hbm.at[pl.ds(ct, blk)], sem_out).wait()

        out_a.wait()
        out_main.wait()

    return edge_scores(pk, ei, ew)
